# Initial kernel scaffold; baseline (speedup 1.0000x reference)
import jax
import jax.numpy as jnp
from jax.experimental import pallas as pl


def kernel(output, mask, ind, rotbin, rotres):
    raise NotImplementedError("write your pallas kernel here")


# speedup vs baseline: 2.4876x; 2.4876x over previous
import functools

import jax
import jax.numpy as jnp
from jax import lax
from jax.experimental import pallas as pl
from jax.experimental.pallas import tpu as pltpu
from jax.experimental.pallas import tpu_sc as plsc

_B, _K, _H, _W = 64, 256, 256, 256
_C = 8
_HW = _H * _W
_NC, _NS, _L = 2, 16, 16
_NW = _NC * _NS
_BPW = _B // _NW
_GPW = _BPW * _K * _C


def _gather_body(outflat_hbm, ind_hbm, pred_hbm, ind_v, idx_v, val_v, sem):
    wid = lax.axis_index("s") * _NC + lax.axis_index("c")
    b0 = wid * _BPW
    pltpu.sync_copy(ind_hbm.at[pl.ds(b0 * _K, _BPW * _K)], ind_v)

    def body(j, carry):
        bl = j // (_K // _L)
        k0 = (j % (_K // _L)) * _L
        ind16 = ind_v[pl.ds(j * _L, _L)]
        base_row = (b0 + bl) * _C
        for c in range(_C):
            idx_v[pl.ds(bl * (_C * _K) + c * _K + k0, _L)] = (
                ind16 + (base_row + c) * _HW
            )
        return carry

    lax.fori_loop(0, _BPW * _K // _L, body, 0)
    pltpu.async_copy(outflat_hbm.at[idx_v], val_v, sem).wait()
    pltpu.sync_copy(val_v, pred_hbm.at[pl.ds(wid * _GPW, _GPW)])


_gather_sc = functools.partial(
    pl.kernel,
    mesh=plsc.VectorSubcoreMesh(core_axis_name="c", subcore_axis_name="s"),
    out_type=jax.ShapeDtypeStruct((_B * _C * _K,), jnp.float32),
    scratch_types=[
        pltpu.VMEM((_BPW * _K,), jnp.int32),
        pltpu.VMEM((_GPW,), jnp.int32),
        pltpu.VMEM((_GPW,), jnp.float32),
        pltpu.SemaphoreType.DMA,
    ],
)(_gather_body)


def _loss_body(pred_ref, rb_ref, tr_ref, mask_ref, out_ref):
    m = mask_ref[...]
    n = jnp.float32(_B * _K)

    def bin_loss(l0, l1, t):
        a = l0 * m
        b = l1 * m
        mx = jnp.maximum(a, b)
        lse = mx + jnp.log(jnp.exp(a - mx) + jnp.exp(b - mx))
        picked = jnp.where(t == 1, b, a)
        return jnp.sum(lse - picked) / n

    def smooth_l1(x, y, msk, cnt):
        d = x - y
        ad = jnp.abs(d)
        v = jnp.where(ad < 1.0, 0.5 * d * d, ad - 0.5)
        s = jnp.sum(v * msk)
        return jnp.where(cnt > 0, s / cnt, 0.0)

    tb0 = rb_ref[0]
    tb1 = rb_ref[1]
    tr0 = tr_ref[0]
    tr1 = tr_ref[1]

    loss_bin1 = bin_loss(pred_ref[:, 0, :], pred_ref[:, 1, :], tb0)
    loss_bin2 = bin_loss(pred_ref[:, 4, :], pred_ref[:, 5, :], tb1)

    m1 = (tb0 != 0).astype(jnp.float32)
    cnt1 = jnp.sum(m1)
    loss_sin1 = smooth_l1(pred_ref[:, 2, :], jnp.sin(tr0), m1, cnt1)
    loss_cos1 = smooth_l1(pred_ref[:, 3, :], jnp.cos(tr0), m1, cnt1)

    m2 = (tb1 != 0).astype(jnp.float32)
    cnt2 = jnp.sum(m2)
    loss_sin2 = smooth_l1(pred_ref[:, 6, :], jnp.sin(tr1), m2, cnt2)
    loss_cos2 = smooth_l1(pred_ref[:, 7, :], jnp.cos(tr1), m2, cnt2)

    total = (
        loss_bin1 + loss_bin2 + loss_sin1 + loss_cos1 + loss_sin2 + loss_cos2
    )
    out_ref[...] = jnp.reshape(total, (1, 1))


def kernel(output, mask, ind, rotbin, rotres):
    pred_flat = _gather_sc(output.reshape(-1), ind.reshape(-1))
    pred = pred_flat.reshape(_B, _C, _K)
    rb_t = jnp.transpose(rotbin, (2, 0, 1))
    tr_t = jnp.transpose(rotres, (2, 0, 1))
    loss = pl.pallas_call(
        _loss_body,
        out_shape=jax.ShapeDtypeStruct((1, 1), jnp.float32),
    )(pred, rb_t, tr_t, mask)
    return loss[0, 0]


# speedup vs baseline: 10.0996x; 4.0600x over previous
import functools

import jax
import jax.numpy as jnp
from jax import lax
from jax.experimental import pallas as pl
from jax.experimental.pallas import tpu as pltpu
from jax.experimental.pallas import tpu_sc as plsc

_B, _K, _H, _W = 64, 256, 256, 256
_C = 8
_NC, _NS, _L = 2, 16, 16
_NW = _NC * _NS
_BPW = _B // _NW
_GPW = _BPW * _K * _C
_KB = _K // _L
_NCHUNK = _BPW * _KB
_RPC = _L * _C


def _gather_body(feat_hbm, ind_hbm, pred_hbm, ind_v, idx_v, val_v, sem):
    wid = lax.axis_index("s") * _NC + lax.axis_index("c")
    b0 = wid * _BPW
    pltpu.sync_copy(ind_hbm.at[pl.ds(b0 * _K, _BPW * _K)], ind_v)

    def body(j, carry):
        bl = j // (_K // _L)
        k0 = (j % (_K // _L)) * _L
        ind16 = ind_v[pl.ds(j * _L, _L)]
        h16 = jnp.right_shift(ind16, 8)
        w16 = jnp.bitwise_and(ind16, _W - 1)
        tiled_off = (
            jnp.left_shift(jnp.right_shift(h16, 3), 11)
            + jnp.left_shift(jnp.right_shift(w16, 7), 10)
            + jnp.left_shift(jnp.bitwise_and(h16, 7), 7)
            + jnp.bitwise_and(w16, 127)
        )
        base_row = (b0 + bl) * _C
        for c in range(_C):
            idx_v[pl.ds(bl * (_C * _K) + c * _K + k0, _L)] = (
                (base_row + c) * (_H * _W) + tiled_off
            )
        return carry

    lax.fori_loop(0, _BPW * _K // _L, body, 0)
    pltpu.async_copy(feat_hbm.at[idx_v], val_v, sem).wait()
    pltpu.sync_copy(val_v, pred_hbm.at[pl.ds(wid * _GPW, _GPW)])


_gather_sc = functools.partial(
    pl.kernel,
    mesh=plsc.VectorSubcoreMesh(core_axis_name="c", subcore_axis_name="s"),
    out_type=jax.ShapeDtypeStruct((_B * _C * _K,), jnp.float32),
    scratch_types=[
        pltpu.VMEM((_BPW * _K,), jnp.int32),
        pltpu.VMEM((_GPW,), jnp.int32),
        pltpu.VMEM((_GPW,), jnp.float32),
        pltpu.SemaphoreType.DMA,
    ],
    compiler_params=pltpu.CompilerParams(use_tc_tiling_on_sc=False),
)(_gather_body)


def _loss_body(pred_ref, rb_ref, tr_ref, mask_ref, out_ref):
    m = mask_ref[...]
    n = jnp.float32(_B * _K)

    def bin_loss(l0, l1, t):
        a = l0 * m
        b = l1 * m
        mx = jnp.maximum(a, b)
        lse = mx + jnp.log(jnp.exp(a - mx) + jnp.exp(b - mx))
        picked = jnp.where(t == 1, b, a)
        return jnp.sum(lse - picked) / n

    def smooth_l1(x, y, msk, cnt):
        d = x - y
        ad = jnp.abs(d)
        v = jnp.where(ad < 1.0, 0.5 * d * d, ad - 0.5)
        s = jnp.sum(v * msk)
        return jnp.where(cnt > 0, s / cnt, 0.0)

    tb0 = rb_ref[0]
    tb1 = rb_ref[1]
    tr0 = tr_ref[0]
    tr1 = tr_ref[1]

    loss_bin1 = bin_loss(pred_ref[:, 0, :], pred_ref[:, 1, :], tb0)
    loss_bin2 = bin_loss(pred_ref[:, 4, :], pred_ref[:, 5, :], tb1)

    m1 = (tb0 != 0).astype(jnp.float32)
    cnt1 = jnp.sum(m1)
    loss_sin1 = smooth_l1(pred_ref[:, 2, :], jnp.sin(tr0), m1, cnt1)
    loss_cos1 = smooth_l1(pred_ref[:, 3, :], jnp.cos(tr0), m1, cnt1)

    m2 = (tb1 != 0).astype(jnp.float32)
    cnt2 = jnp.sum(m2)
    loss_sin2 = smooth_l1(pred_ref[:, 6, :], jnp.sin(tr1), m2, cnt2)
    loss_cos2 = smooth_l1(pred_ref[:, 7, :], jnp.cos(tr1), m2, cnt2)

    total = (
        loss_bin1 + loss_bin2 + loss_sin1 + loss_cos1 + loss_sin2 + loss_cos2
    )
    out_ref[...] = jnp.reshape(total, (1, 1))


def kernel(output, mask, ind, rotbin, rotres):
    feat = (
        output.reshape(_B, _C, _H // 8, 8, _W // 128, 128)
        .transpose(0, 1, 2, 4, 3, 5)
        .reshape(-1)
    )
    pred_flat = _gather_sc(feat, ind.reshape(-1))
    pred = pred_flat.reshape(_B, _C, _K)
    rb_t = jnp.transpose(rotbin, (2, 0, 1))
    tr_t = jnp.transpose(rotres, (2, 0, 1))
    loss = pl.pallas_call(
        _loss_body,
        out_shape=jax.ShapeDtypeStruct((1, 1), jnp.float32),
    )(pred, rb_t, tr_t, mask)
    return loss[0, 0]


# speedup vs baseline: 10.7496x; 1.0644x over previous
import functools

import jax
import jax.numpy as jnp
from jax import lax
from jax.experimental import pallas as pl
from jax.experimental.pallas import tpu as pltpu
from jax.experimental.pallas import tpu_sc as plsc

_B, _K, _H, _W = 64, 256, 256, 256
_C = 8
_NC, _NS, _L = 2, 16, 16
_NW = _NC * _NS
_BPW = _B // _NW
_GPW = _BPW * _K * _C
_KB = _K // _L
_NCHUNK = _BPW * _KB
_RPC = _L * _C


def _gather_body(feat_hbm, ind_hbm, pred_hbm, ind_v, idx_v, val_v, sem):
    wid = lax.axis_index("s") * _NC + lax.axis_index("c")
    b0 = wid * _BPW
    pltpu.sync_copy(ind_hbm.at[pl.ds(b0 * _K, _BPW * _K)], ind_v)

    def build_quarter(q):
        bl = q // 2
        kt = q % 2
        for jj in range(8):
            k0 = kt * 128 + jj * _L
            ind16 = ind_v[pl.ds(bl * _K + k0, _L)]
            h16 = jnp.right_shift(ind16, 8)
            w16 = jnp.bitwise_and(ind16, _W - 1)
            tiled_off = (
                jnp.left_shift(jnp.right_shift(h16, 3), 11)
                + jnp.left_shift(jnp.right_shift(w16, 7), 10)
                + jnp.left_shift(jnp.bitwise_and(h16, 7), 7)
                + jnp.bitwise_and(w16, 127)
            )
            base_row = (b0 + bl) * _C
            for c in range(_C):
                idx_v[pl.ds(q * 1024 + c * 128 + jj * _L, _L)] = (
                    (base_row + c) * (_H * _W) + tiled_off
                )

    for q in range(4):
        build_quarter(q)
        pltpu.async_copy(
            feat_hbm.at[idx_v.at[pl.ds(q * 1024, 1024)]],
            val_v.at[pl.ds(q * 1024, 1024)],
            sem,
        )
    for q in range(4):
        pltpu.make_async_copy(
            feat_hbm.at[idx_v.at[pl.ds(q * 1024, 1024)]],
            val_v.at[pl.ds(q * 1024, 1024)],
            sem,
        ).wait()
    pltpu.sync_copy(val_v, pred_hbm.at[pl.ds(wid * _GPW, _GPW)])


_gather_sc = functools.partial(
    pl.kernel,
    mesh=plsc.VectorSubcoreMesh(core_axis_name="c", subcore_axis_name="s"),
    out_type=jax.ShapeDtypeStruct((_B * _C * _K,), jnp.float32),
    scratch_types=[
        pltpu.VMEM((_BPW * _K,), jnp.int32),
        pltpu.VMEM((_GPW,), jnp.int32),
        pltpu.VMEM((_GPW,), jnp.float32),
        pltpu.SemaphoreType.DMA,
    ],
    compiler_params=pltpu.CompilerParams(use_tc_tiling_on_sc=False),
)(_gather_body)


def _loss_body(pred_ref, rb_ref, tr_ref, mask_ref, out_ref):
    m = mask_ref[...]
    n = jnp.float32(_B * _K)

    def bin_loss(l0, l1, t):
        a = l0 * m
        b = l1 * m
        mx = jnp.maximum(a, b)
        lse = mx + jnp.log(jnp.exp(a - mx) + jnp.exp(b - mx))
        picked = jnp.where(t == 1, b, a)
        return jnp.sum(lse - picked) / n

    def smooth_l1(x, y, msk, cnt):
        d = x - y
        ad = jnp.abs(d)
        v = jnp.where(ad < 1.0, 0.5 * d * d, ad - 0.5)
        s = jnp.sum(v * msk)
        return jnp.where(cnt > 0, s / cnt, 0.0)

    tb0 = rb_ref[0]
    tb1 = rb_ref[1]
    tr0 = tr_ref[0]
    tr1 = tr_ref[1]

    loss_bin1 = bin_loss(pred_ref[:, 0, :], pred_ref[:, 1, :], tb0)
    loss_bin2 = bin_loss(pred_ref[:, 4, :], pred_ref[:, 5, :], tb1)

    m1 = (tb0 != 0).astype(jnp.float32)
    cnt1 = jnp.sum(m1)
    loss_sin1 = smooth_l1(pred_ref[:, 2, :], jnp.sin(tr0), m1, cnt1)
    loss_cos1 = smooth_l1(pred_ref[:, 3, :], jnp.cos(tr0), m1, cnt1)

    m2 = (tb1 != 0).astype(jnp.float32)
    cnt2 = jnp.sum(m2)
    loss_sin2 = smooth_l1(pred_ref[:, 6, :], jnp.sin(tr1), m2, cnt2)
    loss_cos2 = smooth_l1(pred_ref[:, 7, :], jnp.cos(tr1), m2, cnt2)

    total = (
        loss_bin1 + loss_bin2 + loss_sin1 + loss_cos1 + loss_sin2 + loss_cos2
    )
    out_ref[...] = jnp.reshape(total, (1, 1))


def kernel(output, mask, ind, rotbin, rotres):
    feat = (
        output.reshape(_B, _C, _H // 8, 8, _W // 128, 128)
        .transpose(0, 1, 2, 4, 3, 5)
        .reshape(-1)
    )
    pred_flat = _gather_sc(feat, ind.reshape(-1))
    pred = (
        pred_flat.reshape(_B, 2, _C, 128)
        .transpose(0, 2, 1, 3)
        .reshape(_B, _C, _K)
    )
    rb_t = jnp.transpose(rotbin, (2, 0, 1))
    tr_t = jnp.transpose(rotres, (2, 0, 1))
    loss = pl.pallas_call(
        _loss_body,
        out_shape=jax.ShapeDtypeStruct((1, 1), jnp.float32),
    )(pred, rb_t, tr_t, mask)
    return loss[0, 0]


# speedup vs baseline: 10.7513x; 1.0002x over previous
import functools

import jax
import jax.numpy as jnp
from jax import lax
from jax.experimental import pallas as pl
from jax.experimental.pallas import tpu as pltpu
from jax.experimental.pallas import tpu_sc as plsc

_B, _K, _H, _W = 64, 256, 256, 256
_C = 8
_NC, _NS, _L = 2, 16, 16
_NW = _NC * _NS
_BPW = _B // _NW
_GPW = _BPW * _K * _C
_KB = _K // _L
_NCHUNK = _BPW * _KB
_RPC = _L * _C


def _gather_body(feat_hbm, ind_hbm, pred_hbm, ind_v, idx_v, val_v, sem):
    wid = lax.axis_index("s") * _NC + lax.axis_index("c")
    b0 = wid * _BPW
    pltpu.sync_copy(ind_hbm.at[pl.ds(b0 * _K, _BPW * _K)], ind_v)

    def build_quarter(q):
        bl = q // 2
        kt = q % 2
        for jj in range(8):
            k0 = kt * 128 + jj * _L
            ind16 = ind_v[pl.ds(bl * _K + k0, _L)]
            h16 = jnp.right_shift(ind16, 8)
            w16 = jnp.bitwise_and(ind16, _W - 1)
            tiled_off = (
                jnp.left_shift(jnp.right_shift(h16, 3), 11)
                + jnp.left_shift(jnp.right_shift(w16, 7), 10)
                + jnp.left_shift(jnp.bitwise_and(h16, 7), 7)
                + jnp.bitwise_and(w16, 127)
            )
            base_row = (b0 + bl) * _C
            for c in range(_C):
                idx_v[pl.ds(q * 1024 + c * 128 + jj * _L, _L)] = (
                    (base_row + c) * (_H * _W) + tiled_off
                )

    for q in range(4):
        build_quarter(q)
        pltpu.async_copy(
            feat_hbm.at[idx_v.at[pl.ds(q * 1024, 1024)]],
            val_v.at[pl.ds(q * 1024, 1024)],
            sem,
        )
    for q in range(4):
        pltpu.make_async_copy(
            feat_hbm.at[idx_v.at[pl.ds(q * 1024, 1024)]],
            val_v.at[pl.ds(q * 1024, 1024)],
            sem,
        ).wait()
    pltpu.sync_copy(val_v, pred_hbm.at[pl.ds(wid * _GPW, _GPW)])


_gather_sc = functools.partial(
    pl.kernel,
    mesh=plsc.VectorSubcoreMesh(core_axis_name="c", subcore_axis_name="s"),
    out_type=jax.ShapeDtypeStruct((_B * _C * _K,), jnp.float32),
    scratch_types=[
        pltpu.VMEM((_BPW * _K,), jnp.int32),
        pltpu.VMEM((_GPW,), jnp.int32),
        pltpu.VMEM((_GPW,), jnp.float32),
        pltpu.SemaphoreType.DMA,
    ],
    compiler_params=pltpu.CompilerParams(use_tc_tiling_on_sc=False),
)(_gather_body)


def _loss_body(pred_ref, rb_ref, tr_ref, mask_ref, out_ref):
    m = mask_ref[...]
    n = jnp.float32(_B * _K)

    def ce_mean(l0, l1, t):
        a = l0 * m
        b = l1 * m
        mx = jnp.maximum(a, b)
        lse = mx + jnp.log(1.0 + jnp.exp(-jnp.abs(a - b)))
        tf = t.astype(jnp.float32)
        picked = a + tf * (b - a)
        return jnp.sum(lse - picked) / n

    def sl1_sum(x, y, msk):
        d = x - y
        ad = jnp.abs(d)
        v = jnp.where(ad < 1.0, 0.5 * d * d, ad - 0.5)
        return jnp.sum(v * msk)

    tb0 = rb_ref[0]
    tb1 = rb_ref[1]
    tr0 = tr_ref[0]
    tr1 = tr_ref[1]
    m1 = (tb0 != 0).astype(jnp.float32)
    m2 = (tb1 != 0).astype(jnp.float32)
    cnt1 = jnp.sum(m1)
    cnt2 = jnp.sum(m2)

    loss_bin = ce_mean(pred_ref[:, 0, :], pred_ref[:, 1, :], tb0) + ce_mean(
        pred_ref[:, 4, :], pred_ref[:, 5, :], tb1
    )
    res1 = sl1_sum(pred_ref[:, 2, :], jnp.sin(tr0), m1) + sl1_sum(
        pred_ref[:, 3, :], jnp.cos(tr0), m1
    )
    res2 = sl1_sum(pred_ref[:, 6, :], jnp.sin(tr1), m2) + sl1_sum(
        pred_ref[:, 7, :], jnp.cos(tr1), m2
    )
    total = (
        loss_bin
        + jnp.where(cnt1 > 0, res1 / cnt1, 0.0)
        + jnp.where(cnt2 > 0, res2 / cnt2, 0.0)
    )
    out_ref[...] = jnp.reshape(total, (1, 1))


def kernel(output, mask, ind, rotbin, rotres):
    feat = (
        output.reshape(_B, _C, _H // 8, 8, _W // 128, 128)
        .transpose(0, 1, 2, 4, 3, 5)
        .reshape(-1)
    )
    pred_flat = _gather_sc(feat, ind.reshape(-1))
    pred = (
        pred_flat.reshape(_B, 2, _C, 128)
        .transpose(0, 2, 1, 3)
        .reshape(_B, _C, _K)
    )
    rb_t = jnp.transpose(rotbin, (2, 0, 1))
    tr_t = jnp.transpose(rotres, (2, 0, 1))
    loss = pl.pallas_call(
        _loss_body,
        out_shape=jax.ShapeDtypeStruct((1, 1), jnp.float32),
    )(pred, rb_t, tr_t, mask)
    return loss[0, 0]


# speedup vs baseline: 10.7756x; 1.0023x over previous
import functools

import jax
import jax.numpy as jnp
from jax import lax
from jax.experimental import pallas as pl
from jax.experimental.pallas import tpu as pltpu
from jax.experimental.pallas import tpu_sc as plsc

_B, _K, _H, _W = 64, 256, 256, 256
_C = 8
_NC, _NS, _L = 2, 16, 16
_NW = _NC * _NS
_BPW = _B // _NW
_GPW = _BPW * _K * _C


def _gather_body(feat_hbm, ind_hbm, pred_hbm, ind_v, idx_v, val_v, sem):
    wid = lax.axis_index("s") * _NC + lax.axis_index("c")
    b0 = wid * _BPW
    pltpu.sync_copy(ind_hbm.at[pl.ds(b0 * _K, _BPW * _K)], ind_v)

    def build_quarter(q):
        bl = q // 2
        kt = q % 2
        for jj in range(8):
            k0 = kt * 128 + jj * _L
            ind16 = ind_v[pl.ds(bl * _K + k0, _L)]
            h16 = jnp.right_shift(ind16, 8)
            w16 = jnp.bitwise_and(ind16, _W - 1)
            tiled_off = (
                jnp.left_shift(jnp.right_shift(h16, 3), 11)
                + jnp.left_shift(jnp.right_shift(w16, 7), 10)
                + jnp.left_shift(jnp.bitwise_and(h16, 7), 7)
                + jnp.bitwise_and(w16, 127)
            )
            base_row = (b0 + bl) * _C
            for c in range(_C):
                idx_v[pl.ds(q * 1024 + c * 128 + jj * _L, _L)] = (
                    (base_row + c) * (_H * _W) + tiled_off
                )

    for q in range(4):
        build_quarter(q)
        pltpu.async_copy(
            feat_hbm.at[idx_v.at[pl.ds(q * 1024, 1024)]],
            val_v.at[pl.ds(q * 1024, 1024)],
            sem,
        )
    for q in range(4):
        pltpu.make_async_copy(
            feat_hbm.at[idx_v.at[pl.ds(q * 1024, 1024)]],
            val_v.at[pl.ds(q * 1024, 1024)],
            sem,
        ).wait()
    pltpu.sync_copy(val_v, pred_hbm.at[pl.ds(wid * _GPW, _GPW)])


_gather_sc = functools.partial(
    pl.kernel,
    mesh=plsc.VectorSubcoreMesh(core_axis_name="c", subcore_axis_name="s"),
    out_type=jax.ShapeDtypeStruct((_B * _C * _K,), jnp.float32),
    scratch_types=[
        pltpu.VMEM((_BPW * _K,), jnp.int32),
        pltpu.VMEM((_GPW,), jnp.int32),
        pltpu.VMEM((_GPW,), jnp.float32),
        pltpu.SemaphoreType.DMA,
    ],
    compiler_params=pltpu.CompilerParams(use_tc_tiling_on_sc=False),
)(_gather_body)


def _loss_body(pred_ref, rb_ref, tr_ref, mask_ref, out_ref):
    m = mask_ref[...]
    n = jnp.float32(_B * _K)

    def ce_mean(l0, l1, t):
        a = l0 * m
        b = l1 * m
        mx = jnp.maximum(a, b)
        lse = mx + jnp.log(1.0 + jnp.exp(-jnp.abs(a - b)))
        tf = t.astype(jnp.float32)
        picked = a + tf * (b - a)
        return jnp.sum(lse - picked) / n

    def sl1_sum(x, y, msk):
        d = x - y
        ad = jnp.abs(d)
        v = jnp.where(ad < 1.0, 0.5 * d * d, ad - 0.5)
        return jnp.sum(v * msk)

    tb0 = rb_ref[0]
    tb1 = rb_ref[1]
    tr0 = tr_ref[0]
    tr1 = tr_ref[1]
    m1 = (tb0 != 0).astype(jnp.float32)
    m2 = (tb1 != 0).astype(jnp.float32)
    cnt1 = jnp.sum(m1)
    cnt2 = jnp.sum(m2)

    loss_bin = ce_mean(pred_ref[:, 0, :], pred_ref[:, 1, :], tb0) + ce_mean(
        pred_ref[:, 4, :], pred_ref[:, 5, :], tb1
    )
    res1 = sl1_sum(pred_ref[:, 2, :], jnp.sin(tr0), m1) + sl1_sum(
        pred_ref[:, 3, :], jnp.cos(tr0), m1
    )
    res2 = sl1_sum(pred_ref[:, 6, :], jnp.sin(tr1), m2) + sl1_sum(
        pred_ref[:, 7, :], jnp.cos(tr1), m2
    )
    total = (
        loss_bin
        + jnp.where(cnt1 > 0, res1 / cnt1, 0.0)
        + jnp.where(cnt2 > 0, res2 / cnt2, 0.0)
    )
    out_ref[...] = jnp.reshape(total, (1, 1))


def kernel(output, mask, ind, rotbin, rotres):
    feat = (
        output.reshape(_B, _C, _H // 8, 8, _W // 128, 128)
        .transpose(0, 1, 2, 4, 3, 5)
        .reshape(-1)
    )
    pred_flat = _gather_sc(feat, ind.reshape(-1))
    pred = (
        pred_flat.reshape(_B, 2, _C, 128)
        .transpose(0, 2, 1, 3)
        .reshape(_B, _C, _K)
    )
    rb_t = jnp.transpose(rotbin, (2, 0, 1))
    tr_t = jnp.transpose(rotres, (2, 0, 1))
    loss = pl.pallas_call(
        _loss_body,
        out_shape=jax.ShapeDtypeStruct((1, 1), jnp.float32),
    )(pred, rb_t, tr_t, mask)
    return loss[0, 0]
